# baseline (speedup 1.0000x reference)
import functools

import jax
import jax.numpy as jnp
import numpy as np
from jax import lax
from jax.experimental import pallas as pl
from jax.experimental.pallas import tpu as pltpu
from jax.experimental.pallas import tpu_sc as plsc

_N = 325
_E = 105300
_B = 8
_T = 12
_D = 2
_H = 64
_HG = 128
_TT = 2016

_NP = 336
_NM = _NP * _NP
_NTILES = 32
_EPT = 3328
_NCH = _EPT // 128
_EP = _EPT * _NTILES

_UU = np.asarray(jax.random.uniform(jax.random.key(42), (_E, 2),
                                    minval=1e-9, maxval=1.0))
_U0P = np.concatenate([_UU[:, 0], np.full(_EP - _E, 0.5, np.float32)])
_U1P = np.concatenate([_UU[:, 1], np.full(_EP - _E, 0.5, np.float32)])


def _tc1_body(ei, wg1, wg2a, wg2b, a_ref, b_ref):
    feat = jnp.maximum(
        jnp.dot(ei[...], wg1[...], preferred_element_type=jnp.float32), 0.0)
    a_ref[...] = jnp.dot(feat, wg2a[...], preferred_element_type=jnp.float32)
    b_ref[...] = jnp.dot(feat, wg2b[...], preferred_element_type=jnp.float32)


_tc1 = pl.pallas_call(
    _tc1_body,
    out_shape=[
        jax.ShapeDtypeStruct((_NP, 2), jnp.float32),
        jax.ShapeDtypeStruct((_NP, 2), jnp.float32),
    ],
)


_sc_mesh = plsc.VectorSubcoreMesh(
    core_axis_name="c", subcore_axis_name="s", num_cores=2, num_subcores=16)


@functools.partial(
    pl.kernel,
    out_type=[
        jax.ShapeDtypeStruct((_EP,), jnp.float32),
        jax.ShapeDtypeStruct((2, _NM), jnp.float32),
    ],
    mesh=_sc_mesh,
    compiler_params=pltpu.CompilerParams(needs_layout_passes=False),
    scratch_types=[
        pltpu.VMEM((_EPT,), jnp.int32),
        pltpu.VMEM((_EPT,), jnp.int32),
        pltpu.VMEM((_EPT,), jnp.float32),
        pltpu.VMEM((_EPT,), jnp.float32),
        pltpu.VMEM((_EPT,), jnp.float32),
        pltpu.VMEM((_NCH, 128), jnp.int32),
        pltpu.VMEM((_NP,), jnp.float32),
        pltpu.VMEM((_NP,), jnp.float32),
        pltpu.VMEM((_NP,), jnp.float32),
        pltpu.VMEM((_NP,), jnp.float32),
        pltpu.VMEM_SHARED((_NM,), jnp.float32),
        pltpu.SemaphoreType.DMA,
        pltpu.SemaphoreType.DMA,
    ],
)
def _sc_edges(src_hbm, dst_hbm, g0_hbm, g1_hbm, a0_hbm, a1_hbm, b0_hbm,
              b1_hbm, z_hbm, mask_hbm, m_hbm,
              src_v, dst_v, g0_v, g1_v, mv_v, id_v, a0_v, a1_v, b0_v, b1_v,
              m_sh, sem_in, sem_sc):
    cid = lax.axis_index("c")
    sid = lax.axis_index("s")
    wid = cid * 16 + sid
    base = wid * _EPT
    pltpu.async_copy(src_hbm.at[pl.ds(base, _EPT)], src_v, sem_in)
    pltpu.async_copy(dst_hbm.at[pl.ds(base, _EPT)], dst_v, sem_in)
    pltpu.async_copy(g0_hbm.at[pl.ds(base, _EPT)], g0_v, sem_in)
    pltpu.async_copy(g1_hbm.at[pl.ds(base, _EPT)], g1_v, sem_in)
    pltpu.async_copy(a0_hbm, a0_v, sem_in)
    pltpu.async_copy(a1_hbm, a1_v, sem_in)
    pltpu.async_copy(b0_hbm, b0_v, sem_in)
    pltpu.async_copy(b1_hbm, b1_v, sem_in)

    @pl.when(sid == 0)
    def _():
        pltpu.sync_copy(z_hbm, m_sh)

    pltpu.make_async_copy(src_hbm.at[pl.ds(base, _EPT)], src_v, sem_in).wait()
    pltpu.make_async_copy(dst_hbm.at[pl.ds(base, _EPT)], dst_v, sem_in).wait()
    pltpu.make_async_copy(g0_hbm.at[pl.ds(base, _EPT)], g0_v, sem_in).wait()
    pltpu.make_async_copy(g1_hbm.at[pl.ds(base, _EPT)], g1_v, sem_in).wait()
    pltpu.make_async_copy(a0_hbm, a0_v, sem_in).wait()
    pltpu.make_async_copy(a1_hbm, a1_v, sem_in).wait()
    pltpu.make_async_copy(b0_hbm, b0_v, sem_in).wait()
    pltpu.make_async_copy(b1_hbm, b1_v, sem_in).wait()

    plsc.subcore_barrier()

    @pl.loop(0, _NCH)
    def _(ci):
        for j in range(8):
            off = ci * 128 + j * 16
            s = src_v[pl.ds(off, 16)]
            d = dst_v[pl.ds(off, 16)]
            x0 = (plsc.load_gather(a0_v, [s]) + plsc.load_gather(b0_v, [d])
                  ) + g0_v[pl.ds(off, 16)]
            x1 = (plsc.load_gather(a1_v, [s]) + plsc.load_gather(b1_v, [d])
                  ) + g1_v[pl.ds(off, 16)]
            m = jnp.where(x0 >= x1, 1.0, 0.0).astype(jnp.float32)
            mv_v[pl.ds(off, 16)] = m
            id_v[ci, pl.ds(j * 16, 16)] = d * _NP + s
        pltpu.async_copy(mv_v.at[pl.ds(ci * 128, 128)], m_sh.at[id_v.at[ci]],
                         sem_sc, add=True)

    pltpu.sync_copy(mv_v, mask_hbm.at[pl.ds(base, _EPT)])

    @pl.loop(0, _NCH)
    def _(ci):
        pltpu.make_async_copy(mv_v.at[pl.ds(ci * 128, 128)],
                              m_sh.at[id_v.at[ci]], sem_sc).wait()

    plsc.subcore_barrier()

    @pl.when(sid == 0)
    def _():
        pltpu.sync_copy(m_sh, m_hbm.at[cid])


def _tc2_body(mr, xin, wenc, wself, wnbr, wout, o_ref):
    m = mr[0] + mr[1]
    h = jnp.dot(xin[...], wenc[...], preferred_element_type=jnp.float32)
    for _ in range(2):
        agg = jnp.dot(m, h, preferred_element_type=jnp.float32)
        h = jnp.maximum(
            jnp.dot(h, wself[...], preferred_element_type=jnp.float32)
            + jnp.dot(agg, wnbr[...], preferred_element_type=jnp.float32),
            0.0)
    o_ref[...] = jnp.dot(h, wout[...], preferred_element_type=jnp.float32)


_tc2 = pl.pallas_call(
    _tc2_body,
    out_shape=jax.ShapeDtypeStruct((_NP, _B * _T * _D), jnp.float32),
)


def kernel(inputs, targets, entire_inputs, edge_index, Wg1, Wg2, Wenc, Wself,
           Wnbr, Wout):
    f32 = jnp.float32
    src = edge_index[0].astype(jnp.int32)
    dst = edge_index[1].astype(jnp.int32)
    pad = _EP - _E
    srcp = jnp.concatenate([src, jnp.full((pad,), _NP - 1, jnp.int32)])
    dstp = jnp.concatenate([dst, jnp.full((pad,), _NP - 1, jnp.int32)])

    g0 = -jnp.log(-jnp.log(jnp.asarray(_U0P)))
    g1 = -jnp.log(-jnp.log(jnp.asarray(_U1P)))

    eip = jnp.pad(entire_inputs, ((0, _NP - _N), (0, 0)))

    ahalf, bhalf = _tc1(eip, Wg1, Wg2[:_HG], Wg2[_HG:])

    zeros = jnp.zeros((_NM,), f32)
    maskp, mraw = _sc_edges(srcp, dstp, g0, g1, ahalf[:, 0], ahalf[:, 1],
                            bhalf[:, 0], bhalf[:, 1], zeros)
    edge_mask = maskp[:_E]

    td = _T * _D
    inp2 = inputs.reshape(_B, _N, td).transpose(1, 0, 2).reshape(_N, _B * td)
    inp2 = jnp.pad(inp2, ((0, _NP - _N), (0, 0)))
    eye = jnp.eye(_B, dtype=f32)
    w8enc = jnp.kron(eye, Wenc)
    w8self = jnp.kron(eye, Wself)
    w8nbr = jnp.kron(eye, Wnbr)
    w8out = jnp.kron(eye, Wout)

    o = _tc2(mraw.reshape(2, _NP, _NP), inp2, w8enc, w8self, w8nbr, w8out)
    outputs = o[:_N].reshape(_N, _B, td).transpose(1, 0, 2).reshape(
        _B * _N, _T, _D)
    return (edge_mask, outputs)


# speedup vs baseline: 1.0363x; 1.0363x over previous
import functools

import jax
import jax.numpy as jnp
import numpy as np
from jax import lax
from jax.experimental import pallas as pl
from jax.experimental.pallas import tpu as pltpu
from jax.experimental.pallas import tpu_sc as plsc

_N = 325
_E = 105300
_B = 8
_T = 12
_D = 2
_H = 64
_HG = 128
_TT = 2016

_NP = 336
_NM = _NP * _NP
_NTILES = 32
_EPT = 3328
_NCH = _EPT // 128
_EP = _EPT * _NTILES

_UU = np.asarray(jax.random.uniform(jax.random.key(42), (_E, 2),
                                    minval=1e-9, maxval=1.0))
_U0P = np.concatenate([_UU[:, 0], np.full(_EP - _E, 0.5, np.float32)])
_U1P = np.concatenate([_UU[:, 1], np.full(_EP - _E, 0.5, np.float32)])


def _tc1_body(ei, wg1, wg2a, wg2b, a_ref, b_ref):
    feat = jnp.maximum(
        jnp.dot(ei[...], wg1[...], preferred_element_type=jnp.float32), 0.0)
    a_ref[: _N, :] = jnp.dot(feat, wg2a[...],
                             preferred_element_type=jnp.float32)
    b_ref[: _N, :] = jnp.dot(feat, wg2b[...],
                             preferred_element_type=jnp.float32)


_tc1 = pl.pallas_call(
    _tc1_body,
    out_shape=[
        jax.ShapeDtypeStruct((_NP, 2), jnp.float32),
        jax.ShapeDtypeStruct((_NP, 2), jnp.float32),
    ],
)


_sc_mesh = plsc.VectorSubcoreMesh(
    core_axis_name="c", subcore_axis_name="s", num_cores=2, num_subcores=16)


@functools.partial(
    pl.kernel,
    out_type=[
        jax.ShapeDtypeStruct((_EP,), jnp.float32),
        jax.ShapeDtypeStruct((2, _NM), jnp.float32),
    ],
    mesh=_sc_mesh,
    compiler_params=pltpu.CompilerParams(needs_layout_passes=False),
    scratch_types=[
        pltpu.VMEM((_EPT,), jnp.int32),
        pltpu.VMEM((_EPT,), jnp.int32),
        pltpu.VMEM((_EPT,), jnp.float32),
        pltpu.VMEM((_EPT,), jnp.float32),
        pltpu.VMEM((_EPT,), jnp.float32),
        pltpu.VMEM((_NCH, 128), jnp.int32),
        pltpu.VMEM((_NP,), jnp.float32),
        pltpu.VMEM((_NP,), jnp.float32),
        pltpu.VMEM((_NP,), jnp.float32),
        pltpu.VMEM((_NP,), jnp.float32),
        pltpu.VMEM_SHARED((_NM,), jnp.float32),
        pltpu.SemaphoreType.DMA,
        pltpu.SemaphoreType.DMA,
    ],
)
def _sc_edges(src_hbm, dst_hbm, g0_hbm, g1_hbm, a0_hbm, a1_hbm, b0_hbm,
              b1_hbm, z_hbm, mask_hbm, m_hbm,
              src_v, dst_v, g0_v, g1_v, mv_v, id_v, a0_v, a1_v, b0_v, b1_v,
              m_sh, sem_in, sem_sc):
    cid = lax.axis_index("c")
    sid = lax.axis_index("s")
    wid = cid * 16 + sid
    base = wid * _EPT
    pltpu.async_copy(src_hbm.at[pl.ds(base, _EPT)], src_v, sem_in)
    pltpu.async_copy(dst_hbm.at[pl.ds(base, _EPT)], dst_v, sem_in)
    pltpu.async_copy(g0_hbm.at[pl.ds(base, _EPT)], g0_v, sem_in)
    pltpu.async_copy(g1_hbm.at[pl.ds(base, _EPT)], g1_v, sem_in)
    pltpu.async_copy(a0_hbm, a0_v, sem_in)
    pltpu.async_copy(a1_hbm, a1_v, sem_in)
    pltpu.async_copy(b0_hbm, b0_v, sem_in)
    pltpu.async_copy(b1_hbm, b1_v, sem_in)

    @pl.when(sid == 0)
    def _():
        pltpu.sync_copy(z_hbm, m_sh)

    pltpu.make_async_copy(src_hbm.at[pl.ds(base, _EPT)], src_v, sem_in).wait()
    pltpu.make_async_copy(dst_hbm.at[pl.ds(base, _EPT)], dst_v, sem_in).wait()
    pltpu.make_async_copy(g0_hbm.at[pl.ds(base, _EPT)], g0_v, sem_in).wait()
    pltpu.make_async_copy(g1_hbm.at[pl.ds(base, _EPT)], g1_v, sem_in).wait()
    pltpu.make_async_copy(a0_hbm, a0_v, sem_in).wait()
    pltpu.make_async_copy(a1_hbm, a1_v, sem_in).wait()
    pltpu.make_async_copy(b0_hbm, b0_v, sem_in).wait()
    pltpu.make_async_copy(b1_hbm, b1_v, sem_in).wait()

    plsc.subcore_barrier()

    @pl.loop(0, _NCH)
    def _(ci):
        for j in range(8):
            off = ci * 128 + j * 16
            s = src_v[pl.ds(off, 16)]
            d = dst_v[pl.ds(off, 16)]
            x0 = (plsc.load_gather(a0_v, [s]) + plsc.load_gather(b0_v, [d])
                  ) + g0_v[pl.ds(off, 16)]
            x1 = (plsc.load_gather(a1_v, [s]) + plsc.load_gather(b1_v, [d])
                  ) + g1_v[pl.ds(off, 16)]
            m = jnp.where(x0 >= x1, 1.0, 0.0).astype(jnp.float32)
            mv_v[pl.ds(off, 16)] = m
            id_v[ci, pl.ds(j * 16, 16)] = d * _NP + s
        pltpu.async_copy(mv_v.at[pl.ds(ci * 128, 128)], m_sh.at[id_v.at[ci]],
                         sem_sc, add=True)

    pltpu.sync_copy(mv_v, mask_hbm.at[pl.ds(base, _EPT)])

    @pl.loop(0, _NCH)
    def _(ci):
        pltpu.make_async_copy(mv_v.at[pl.ds(ci * 128, 128)],
                              m_sh.at[id_v.at[ci]], sem_sc).wait()

    plsc.subcore_barrier()

    @pl.when(sid == 0)
    def _():
        pltpu.sync_copy(m_sh, m_hbm.at[cid])


def _tc2_body(mr, xin, wenc, wself, wnbr, wout, o_ref):
    m = mr[0] + mr[1]
    h = jnp.dot(xin[...], wenc[...], preferred_element_type=jnp.float32)
    for _ in range(2):
        agg = jnp.dot(m, h, preferred_element_type=jnp.float32)
        h = jnp.maximum(
            jnp.dot(h, wself[...], preferred_element_type=jnp.float32)
            + jnp.dot(agg, wnbr[...], preferred_element_type=jnp.float32),
            0.0)
    o_ref[...] = jnp.dot(h, wout[...], preferred_element_type=jnp.float32)


_tc2 = pl.pallas_call(
    _tc2_body,
    out_shape=jax.ShapeDtypeStruct((_NP, _B * _T * _D), jnp.float32),
)


def kernel(inputs, targets, entire_inputs, edge_index, Wg1, Wg2, Wenc, Wself,
           Wnbr, Wout):
    f32 = jnp.float32
    src = edge_index[0].astype(jnp.int32)
    dst = edge_index[1].astype(jnp.int32)
    pad = _EP - _E
    srcp = jnp.concatenate([src, jnp.full((pad,), _NP - 1, jnp.int32)])
    dstp = jnp.concatenate([dst, jnp.full((pad,), _NP - 1, jnp.int32)])

    g0 = -jnp.log(-jnp.log(jnp.asarray(_U0P)))
    g1 = -jnp.log(-jnp.log(jnp.asarray(_U1P)))

    ahalf, bhalf = _tc1(entire_inputs, Wg1, Wg2[:_HG], Wg2[_HG:])

    zeros = jnp.zeros((_NM,), f32)
    maskp, mraw = _sc_edges(srcp, dstp, g0, g1, ahalf[:, 0], ahalf[:, 1],
                            bhalf[:, 0], bhalf[:, 1], zeros)
    edge_mask = maskp[:_E]

    td = _T * _D
    inp2 = inputs.reshape(_B, _N, td).transpose(1, 0, 2).reshape(_N, _B * td)
    inp2 = jnp.pad(inp2, ((0, _NP - _N), (0, 0)))
    eye = jnp.eye(_B, dtype=f32)
    w8enc = jnp.kron(eye, Wenc)
    w8self = jnp.kron(eye, Wself)
    w8nbr = jnp.kron(eye, Wnbr)
    w8out = jnp.kron(eye, Wout)

    o = _tc2(mraw.reshape(2, _NP, _NP), inp2, w8enc, w8self, w8nbr, w8out)
    outputs = o[:_N].reshape(_N, _B, td).transpose(1, 0, 2).reshape(
        _B * _N, _T, _D)
    return (edge_mask, outputs)
